# Initial kernel scaffold; baseline (speedup 1.0000x reference)
import jax
import jax.numpy as jnp
from jax.experimental import pallas as pl


def kernel(v_1, v_2, adj_indices, adj_weights, laplace_w):
    raise NotImplementedError("write your pallas kernel here")


# speedup vs baseline: 36.6181x; 36.6181x over previous
import jax
import jax.numpy as jnp
from jax import lax
from jax.experimental import pallas as pl
from jax.experimental.pallas import tpu as pltpu
from jax.experimental.pallas import tpu_sc as plsc

_N = 27554
_K = 9
_NS = 16
_L = 16
_NTAB = 27648
_CHUNK = _NTAB // _NS
_NV = _CHUNK // _L
_SCALE = 1.0 / (3.0 * _N)


def _lap_body(v1_hbm, v2_hbm, idx_hbm, aw_hbm, lw_hbm, out_hbm,
              bufa, bufb, tabx, taby, tabz, idx_v, aw_v, lw_v,
              stage_v, part_v, spx, spy, spz, sp_part):
    s = lax.axis_index("s")
    base = s * _CHUNK

    for c in range(3):
        pltpu.sync_copy(v1_hbm.at[pl.ds(c * _NTAB + base, _CHUNK)],
                        bufa.at[pl.ds(c * _CHUNK, _CHUNK)])
        pltpu.sync_copy(v2_hbm.at[pl.ds(c * _NTAB + base, _CHUNK)],
                        bufb.at[pl.ds(c * _CHUNK, _CHUNK)])

    def _sub(i, carry):
        off = i * _L
        bufa[pl.ds(off, _L)] = bufa[pl.ds(off, _L)] - bufb[pl.ds(off, _L)]
        return carry

    lax.fori_loop(0, 3 * _NV, _sub, 0, unroll=4)

    pltpu.sync_copy(bufa.at[pl.ds(0, _CHUNK)], spx.at[pl.ds(base, _CHUNK)])
    pltpu.sync_copy(bufa.at[pl.ds(_CHUNK, _CHUNK)], spy.at[pl.ds(base, _CHUNK)])
    pltpu.sync_copy(bufa.at[pl.ds(2 * _CHUNK, _CHUNK)], spz.at[pl.ds(base, _CHUNK)])
    plsc.subcore_barrier()

    pltpu.sync_copy(spx, tabx)
    pltpu.sync_copy(spy, taby)
    pltpu.sync_copy(spz, tabz)
    pltpu.sync_copy(idx_hbm.at[pl.ds(s * _K * _CHUNK, _K * _CHUNK)], idx_v)
    pltpu.sync_copy(aw_hbm.at[pl.ds(base, _CHUNK)], aw_v)
    pltpu.sync_copy(lw_hbm.at[pl.ds(base, _CHUNK)], lw_v)

    def _gather(j, acc):
        off = j * _L
        voff = base + off
        idx0 = idx_v[pl.ds(off, _L)]
        sx = plsc.load_gather(tabx, [idx0])
        sy = plsc.load_gather(taby, [idx0])
        sz = plsc.load_gather(tabz, [idx0])
        for k in range(1, _K):
            idxk = idx_v[pl.ds(k * _CHUNK + off, _L)]
            sx = sx + plsc.load_gather(tabx, [idxk])
            sy = sy + plsc.load_gather(taby, [idxk])
            sz = sz + plsc.load_gather(tabz, [idxk])
        rw = 1.0 / aw_v[pl.ds(off, _L)]
        ex = tabx[pl.ds(voff, _L)] - sx * rw
        ey = taby[pl.ds(voff, _L)] - sy * rw
        ez = tabz[pl.ds(voff, _L)] - sz * rw
        return acc + (ex * ex + ey * ey + ez * ez) * lw_v[pl.ds(off, _L)]

    acc = lax.fori_loop(0, _NV, _gather, jnp.zeros((_L,), jnp.float32))

    stage_v[...] = acc
    pltpu.sync_copy(stage_v, sp_part.at[pl.ds(s * _L, _L)])
    plsc.subcore_barrier()

    @pl.when(s == 0)
    def _():
        pltpu.sync_copy(sp_part, part_v)
        tot = part_v[pl.ds(0, _L)]
        for t in range(1, _NS):
            tot = tot + part_v[pl.ds(t * _L, _L)]
        total = jnp.sum(tot) * _SCALE
        stage_v[...] = jnp.broadcast_to(total, (_L,))
        pltpu.sync_copy(stage_v, out_hbm)


_lap_call = pl.kernel(
    _lap_body,
    out_type=jax.ShapeDtypeStruct((_L,), jnp.float32),
    mesh=plsc.VectorSubcoreMesh(core_axis_name="c", subcore_axis_name="s",
                                num_cores=1),
    compiler_params=pltpu.CompilerParams(needs_layout_passes=False),
    scratch_types=[
        pltpu.VMEM((3 * _CHUNK,), jnp.float32),
        pltpu.VMEM((3 * _CHUNK,), jnp.float32),
        pltpu.VMEM((_NTAB,), jnp.float32),
        pltpu.VMEM((_NTAB,), jnp.float32),
        pltpu.VMEM((_NTAB,), jnp.float32),
        pltpu.VMEM((_K * _CHUNK,), jnp.int32),
        pltpu.VMEM((_CHUNK,), jnp.float32),
        pltpu.VMEM((_CHUNK,), jnp.float32),
        pltpu.VMEM((_L,), jnp.float32),
        pltpu.VMEM((_NS * _L,), jnp.float32),
        pltpu.VMEM_SHARED((_NTAB,), jnp.float32),
        pltpu.VMEM_SHARED((_NTAB,), jnp.float32),
        pltpu.VMEM_SHARED((_NTAB,), jnp.float32),
        pltpu.VMEM_SHARED((_NS * _L,), jnp.float32),
    ],
)


def kernel(v_1, v_2, adj_indices, adj_weights, laplace_w):
    pad = _NTAB - _N
    v1t = jnp.pad(v_1.astype(jnp.float32).T, ((0, 0), (0, pad))).reshape(-1)
    v2t = jnp.pad(v_2.astype(jnp.float32).T, ((0, 0), (0, pad))).reshape(-1)
    idx_t = jnp.pad(adj_indices.astype(jnp.int32)[:, :_K].T, ((0, 0), (0, pad)))
    idx_b = idx_t.reshape(_K, _NS, _CHUNK).transpose(1, 0, 2).reshape(-1)
    aw = jnp.pad(adj_weights[:, 0].astype(jnp.float32), (0, pad),
                 constant_values=1.0)
    lw = jnp.pad(laplace_w[:, 0].astype(jnp.float32), (0, pad),
                 constant_values=0.0)
    out = _lap_call(v1t, v2t, idx_b, aw, lw)
    return out[0]


# speedup vs baseline: 38.7586x; 1.0585x over previous
import jax
import jax.numpy as jnp
from jax import lax
from jax.experimental import pallas as pl
from jax.experimental.pallas import tpu as pltpu
from jax.experimental.pallas import tpu_sc as plsc

_N = 27554
_K = 9
_NC = 2
_NS = 16
_NW = _NC * _NS
_L = 16
_NTAB = 27648
_CHUNK = _NTAB // _NS
_GCH = _NTAB // _NW
_NV = _GCH // _L
_SCALE = 1.0 / (3.0 * _N)


def _lap_body(v1_hbm, v2_hbm, idx_hbm, aw_hbm, lw_hbm, out_hbm,
              bufa, bufb, tabx, taby, tabz, idx_v, aw_v, lw_v,
              stage_v, part_v, spx, spy, spz, sp_part, sem_in, sem_tab):
    c = lax.axis_index("c")
    s = lax.axis_index("s")
    wid = c * _NS + s
    tbase = s * _CHUNK
    gbase = wid * _GCH

    cp_idx = pltpu.async_copy(idx_hbm.at[pl.ds(wid * _K * _GCH, _K * _GCH)],
                              idx_v, sem_in)
    cp_aw = pltpu.async_copy(aw_hbm.at[pl.ds(gbase, _GCH)], aw_v, sem_in)
    cp_lw = pltpu.async_copy(lw_hbm.at[pl.ds(gbase, _GCH)], lw_v, sem_in)

    cps = []
    for cc in range(3):
        cps.append(pltpu.async_copy(
            v1_hbm.at[pl.ds(cc * _NTAB + tbase, _CHUNK)],
            bufa.at[pl.ds(cc * _CHUNK, _CHUNK)], sem_tab))
        cps.append(pltpu.async_copy(
            v2_hbm.at[pl.ds(cc * _NTAB + tbase, _CHUNK)],
            bufb.at[pl.ds(cc * _CHUNK, _CHUNK)], sem_tab))
    for cp in cps:
        cp.wait()

    def _sub(i, carry):
        off = i * _L
        bufa[pl.ds(off, _L)] = bufa[pl.ds(off, _L)] - bufb[pl.ds(off, _L)]
        return carry

    lax.fori_loop(0, 3 * _CHUNK // _L, _sub, 0, unroll=4)

    cps = [pltpu.async_copy(bufa.at[pl.ds(0, _CHUNK)],
                            spx.at[pl.ds(tbase, _CHUNK)], sem_tab),
           pltpu.async_copy(bufa.at[pl.ds(_CHUNK, _CHUNK)],
                            spy.at[pl.ds(tbase, _CHUNK)], sem_tab),
           pltpu.async_copy(bufa.at[pl.ds(2 * _CHUNK, _CHUNK)],
                            spz.at[pl.ds(tbase, _CHUNK)], sem_tab)]
    for cp in cps:
        cp.wait()
    plsc.subcore_barrier()

    cps = [pltpu.async_copy(spx, tabx, sem_tab),
           pltpu.async_copy(spy, taby, sem_tab),
           pltpu.async_copy(spz, tabz, sem_tab)]
    for cp in cps:
        cp.wait()
    cp_idx.wait()
    cp_aw.wait()
    cp_lw.wait()

    def _gather(j, acc):
        off = j * _L
        voff = gbase + off
        idx0 = idx_v[pl.ds(off, _L)]
        sx = plsc.load_gather(tabx, [idx0])
        sy = plsc.load_gather(taby, [idx0])
        sz = plsc.load_gather(tabz, [idx0])
        for k in range(1, _K):
            idxk = idx_v[pl.ds(k * _GCH + off, _L)]
            sx = sx + plsc.load_gather(tabx, [idxk])
            sy = sy + plsc.load_gather(taby, [idxk])
            sz = sz + plsc.load_gather(tabz, [idxk])
        rw = 1.0 / aw_v[pl.ds(off, _L)]
        ex = tabx[pl.ds(voff, _L)] - sx * rw
        ey = taby[pl.ds(voff, _L)] - sy * rw
        ez = tabz[pl.ds(voff, _L)] - sz * rw
        return acc + (ex * ex + ey * ey + ez * ez) * lw_v[pl.ds(off, _L)]

    acc = lax.fori_loop(0, _NV, _gather, jnp.zeros((_L,), jnp.float32))

    stage_v[...] = acc
    pltpu.sync_copy(stage_v, sp_part.at[pl.ds(s * _L, _L)])
    plsc.subcore_barrier()

    @pl.when(s == 0)
    def _():
        pltpu.sync_copy(sp_part, part_v)
        tot = part_v[pl.ds(0, _L)]
        for t in range(1, _NS):
            tot = tot + part_v[pl.ds(t * _L, _L)]
        total = jnp.sum(tot) * _SCALE
        stage_v[...] = jnp.broadcast_to(total, (_L,))
        pltpu.sync_copy(stage_v.at[pl.ds(0, 8)], out_hbm.at[pl.ds(c * 8, 8)])


_lap_call = pl.kernel(
    _lap_body,
    out_type=jax.ShapeDtypeStruct((_L,), jnp.float32),
    mesh=plsc.VectorSubcoreMesh(core_axis_name="c", subcore_axis_name="s",
                                num_cores=_NC),
    compiler_params=pltpu.CompilerParams(needs_layout_passes=False),
    scratch_types=[
        pltpu.VMEM((3 * _CHUNK,), jnp.float32),
        pltpu.VMEM((3 * _CHUNK,), jnp.float32),
        pltpu.VMEM((_NTAB,), jnp.float32),
        pltpu.VMEM((_NTAB,), jnp.float32),
        pltpu.VMEM((_NTAB,), jnp.float32),
        pltpu.VMEM((_K * _GCH,), jnp.int32),
        pltpu.VMEM((_GCH,), jnp.float32),
        pltpu.VMEM((_GCH,), jnp.float32),
        pltpu.VMEM((_L,), jnp.float32),
        pltpu.VMEM((_NS * _L,), jnp.float32),
        pltpu.VMEM_SHARED((_NTAB,), jnp.float32),
        pltpu.VMEM_SHARED((_NTAB,), jnp.float32),
        pltpu.VMEM_SHARED((_NTAB,), jnp.float32),
        pltpu.VMEM_SHARED((_NS * _L,), jnp.float32),
        pltpu.SemaphoreType.DMA,
        pltpu.SemaphoreType.DMA,
    ],
)


def kernel(v_1, v_2, adj_indices, adj_weights, laplace_w):
    pad = _NTAB - _N
    v1t = jnp.pad(v_1.astype(jnp.float32).T, ((0, 0), (0, pad))).reshape(-1)
    v2t = jnp.pad(v_2.astype(jnp.float32).T, ((0, 0), (0, pad))).reshape(-1)
    idx_t = jnp.pad(adj_indices.astype(jnp.int32)[:, :_K].T, ((0, 0), (0, pad)))
    idx_b = idx_t.reshape(_K, _NW, _GCH).transpose(1, 0, 2).reshape(-1)
    aw = jnp.pad(adj_weights[:, 0].astype(jnp.float32), (0, pad),
                 constant_values=1.0)
    lw = jnp.pad(laplace_w[:, 0].astype(jnp.float32), (0, pad),
                 constant_values=0.0)
    out = _lap_call(v1t, v2t, idx_b, aw, lw)
    return out[0] + out[8]


# speedup vs baseline: 40.5006x; 1.0449x over previous
import jax
import jax.numpy as jnp
from jax import lax
from jax.experimental import pallas as pl
from jax.experimental.pallas import tpu as pltpu
from jax.experimental.pallas import tpu_sc as plsc

_N = 27554
_K = 9
_NC = 2
_NS = 16
_NW = _NC * _NS
_L = 16
_NTAB = 27648
_CHUNK = _NTAB // _NS
_GCH = _NTAB // _NW
_NV = _GCH // _L
_SCALE = 1.0 / (3.0 * _N)

_OFF_V1 = 0
_OFF_V2 = 3 * _NTAB
_OFF_AW = 6 * _NTAB
_OFF_LW = 7 * _NTAB
_OFF_IDX = 8 * _NTAB
_FBUF = 17 * _NTAB


def _lap_body(fbuf_hbm, out_hbm,
              bufa, bufb, tab, idx_v, aw_v, lw_v,
              stage_v, part_v, spall, sp_part, sem_in, sem_tab):
    c = lax.axis_index("c")
    s = lax.axis_index("s")
    wid = c * _NS + s
    tbase = s * _CHUNK
    gbase = wid * _GCH

    cp_idx = pltpu.async_copy(
        fbuf_hbm.at[pl.ds(_OFF_IDX + wid * _K * _GCH, _K * _GCH)],
        idx_v, sem_in)
    cp_aw = pltpu.async_copy(fbuf_hbm.at[pl.ds(_OFF_AW + gbase, _GCH)],
                             aw_v, sem_in)
    cp_lw = pltpu.async_copy(fbuf_hbm.at[pl.ds(_OFF_LW + gbase, _GCH)],
                             lw_v, sem_in)

    cps = []
    for cc in range(3):
        cps.append(pltpu.async_copy(
            fbuf_hbm.at[pl.ds(_OFF_V1 + cc * _NTAB + tbase, _CHUNK)],
            bufa.at[pl.ds(cc * _CHUNK, _CHUNK)], sem_tab))
        cps.append(pltpu.async_copy(
            fbuf_hbm.at[pl.ds(_OFF_V2 + cc * _NTAB + tbase, _CHUNK)],
            bufb.at[pl.ds(cc * _CHUNK, _CHUNK)], sem_tab))
    for cp in cps:
        cp.wait()

    def _sub(i, carry):
        off = i * _L
        bufa[pl.ds(off, _L)] = bufa[pl.ds(off, _L)] - bufb[pl.ds(off, _L)]
        return carry

    lax.fori_loop(0, 3 * _CHUNK // _L, _sub, 0, unroll=4)

    cps = [pltpu.async_copy(bufa.at[pl.ds(cc * _CHUNK, _CHUNK)],
                            spall.at[pl.ds(cc * _NTAB + tbase, _CHUNK)],
                            sem_tab)
           for cc in range(3)]
    for cp in cps:
        cp.wait()
    plsc.subcore_barrier()

    pltpu.sync_copy(spall, tab)
    cp_idx.wait()
    cp_aw.wait()
    cp_lw.wait()

    def _gather(j, acc):
        off = j * _L
        voff = gbase + off
        i0 = plsc.bitcast(idx_v[pl.ds(off, _L)], jnp.int32)
        sx = plsc.load_gather(tab, [i0])
        sy = plsc.load_gather(tab, [i0 + _NTAB])
        sz = plsc.load_gather(tab, [i0 + 2 * _NTAB])
        for k in range(1, _K):
            ik = plsc.bitcast(idx_v[pl.ds(k * _GCH + off, _L)], jnp.int32)
            sx = sx + plsc.load_gather(tab, [ik])
            sy = sy + plsc.load_gather(tab, [ik + _NTAB])
            sz = sz + plsc.load_gather(tab, [ik + 2 * _NTAB])
        rw = 1.0 / aw_v[pl.ds(off, _L)]
        ex = tab[pl.ds(voff, _L)] - sx * rw
        ey = tab[pl.ds(_NTAB + voff, _L)] - sy * rw
        ez = tab[pl.ds(2 * _NTAB + voff, _L)] - sz * rw
        return acc + (ex * ex + ey * ey + ez * ez) * lw_v[pl.ds(off, _L)]

    acc = lax.fori_loop(0, _NV, _gather, jnp.zeros((_L,), jnp.float32))

    stage_v[...] = acc
    pltpu.sync_copy(stage_v, sp_part.at[pl.ds(s * _L, _L)])
    plsc.subcore_barrier()

    @pl.when(s == 0)
    def _():
        pltpu.sync_copy(sp_part, part_v)
        tot = part_v[pl.ds(0, _L)]
        for t in range(1, _NS):
            tot = tot + part_v[pl.ds(t * _L, _L)]
        total = jnp.sum(tot) * _SCALE
        stage_v[...] = jnp.broadcast_to(total, (_L,))
        pltpu.sync_copy(stage_v.at[pl.ds(0, 8)], out_hbm.at[pl.ds(c * 8, 8)])


_lap_call = pl.kernel(
    _lap_body,
    out_type=jax.ShapeDtypeStruct((_L,), jnp.float32),
    mesh=plsc.VectorSubcoreMesh(core_axis_name="c", subcore_axis_name="s",
                                num_cores=_NC),
    compiler_params=pltpu.CompilerParams(needs_layout_passes=False),
    scratch_types=[
        pltpu.VMEM((3 * _CHUNK,), jnp.float32),
        pltpu.VMEM((3 * _CHUNK,), jnp.float32),
        pltpu.VMEM((3 * _NTAB,), jnp.float32),
        pltpu.VMEM((_K * _GCH,), jnp.float32),
        pltpu.VMEM((_GCH,), jnp.float32),
        pltpu.VMEM((_GCH,), jnp.float32),
        pltpu.VMEM((_L,), jnp.float32),
        pltpu.VMEM((_NS * _L,), jnp.float32),
        pltpu.VMEM_SHARED((3 * _NTAB,), jnp.float32),
        pltpu.VMEM_SHARED((_NS * _L,), jnp.float32),
        pltpu.SemaphoreType.DMA,
        pltpu.SemaphoreType.DMA,
    ],
)


def kernel(v_1, v_2, adj_indices, adj_weights, laplace_w):
    pad = _NTAB - _N
    v1t = jnp.pad(v_1.astype(jnp.float32).T, ((0, 0), (0, pad))).reshape(-1)
    v2t = jnp.pad(v_2.astype(jnp.float32).T, ((0, 0), (0, pad))).reshape(-1)
    idx_t = jnp.pad(adj_indices.astype(jnp.int32)[:, :_K].T, ((0, 0), (0, pad)))
    idx_b = idx_t.reshape(_K, _NW, _GCH).transpose(1, 0, 2).reshape(-1)
    idx_f = jax.lax.bitcast_convert_type(idx_b, jnp.float32)
    aw = jnp.pad(adj_weights[:, 0].astype(jnp.float32), (0, pad),
                 constant_values=1.0)
    lw = jnp.pad(laplace_w[:, 0].astype(jnp.float32), (0, pad),
                 constant_values=0.0)
    fbuf = jnp.concatenate([v1t, v2t, aw, lw, idx_f])
    out = _lap_call(fbuf)
    return out[0] + out[8]
